# baseline (speedup 1.0000x reference)
import functools

import jax
import jax.numpy as jnp
from jax import lax
from jax.experimental import pallas as pl
from jax.experimental.pallas import tpu as pltpu
from jax.experimental.pallas import tpu_sc as plsc

_BATCH = 16384
_ED = 64
_N_ACT = 128

_NC = 2
_NS = 16
_NW = _NC * _NS
_ROWS_PER_W = _BATCH // _NW
_CHUNK = 128
_NCHUNK = _ROWS_PER_W // _CHUNK


def _sc_gather_body(u_idx, i_idx, ut, it, u_out, i_out,
                    uidx_v, iidx_v, urows_v, irows_v, usem, isem):
    wid = lax.axis_index("s") * _NC + lax.axis_index("c")
    crow = wid * _NCHUNK
    pltpu.sync_copy(u_idx.at[pl.ds(crow, _NCHUNK)], uidx_v)
    pltpu.sync_copy(i_idx.at[pl.ds(crow, _NCHUNK)], iidx_v)
    waits = []
    for c in range(_NCHUNK):
        dst = pl.ds(c * _CHUNK, _CHUNK)
        waits.append(pltpu.async_copy(ut.at[uidx_v.at[c]], urows_v.at[dst], usem))
        waits.append(pltpu.async_copy(it.at[iidx_v.at[c]], irows_v.at[dst], isem))
    for w in waits:
        w.wait()
    base = wid * _ROWS_PER_W
    pltpu.sync_copy(urows_v, u_out.at[pl.ds(base, _ROWS_PER_W)])
    pltpu.sync_copy(irows_v, i_out.at[pl.ds(base, _ROWS_PER_W)])


@functools.cache
def _sc_gather():
    return functools.partial(
        pl.kernel,
        out_type=[
            jax.ShapeDtypeStruct((_BATCH, _ED), jnp.float32),
            jax.ShapeDtypeStruct((_BATCH, _ED), jnp.float32),
        ],
        mesh=plsc.VectorSubcoreMesh(core_axis_name="c", subcore_axis_name="s"),
        compiler_params=pltpu.CompilerParams(use_tc_tiling_on_sc=False),
        scratch_types=[
            pltpu.VMEM((_NCHUNK, _CHUNK), jnp.int32),
            pltpu.VMEM((_NCHUNK, _CHUNK), jnp.int32),
            pltpu.VMEM((_ROWS_PER_W, _ED), jnp.float32),
            pltpu.VMEM((_ROWS_PER_W, _ED), jnp.float32),
            pltpu.SemaphoreType.DMA,
            pltpu.SemaphoreType.DMA,
        ],
    )(_sc_gather_body)


_BLK = 2048


def _mlp_body(u_ref, i_ref, w1u_ref, w1i_ref, b1_ref, w2_ref, b2_ref, o_ref):
    h = jnp.dot(u_ref[...], w1u_ref[...], preferred_element_type=jnp.float32)
    h += jnp.dot(i_ref[...], w1i_ref[...], preferred_element_type=jnp.float32)
    h = jnp.maximum(h + b1_ref[...], 0.0)
    p = jnp.sum(h * w2_ref[...], axis=1, keepdims=True) + b2_ref[...]
    o_ref[...] = 5.0 / (1.0 + jnp.exp(-p))


_mlp = pl.pallas_call(
    _mlp_body,
    grid=(_BATCH // _BLK,),
    in_specs=[
        pl.BlockSpec((_BLK, _ED), lambda i: (i, 0)),
        pl.BlockSpec((_BLK, _ED), lambda i: (i, 0)),
        pl.BlockSpec((_ED, _N_ACT), lambda i: (0, 0)),
        pl.BlockSpec((_ED, _N_ACT), lambda i: (0, 0)),
        pl.BlockSpec((1, _N_ACT), lambda i: (0, 0)),
        pl.BlockSpec((1, _N_ACT), lambda i: (0, 0)),
        pl.BlockSpec((1, 1), lambda i: (0, 0)),
    ],
    out_specs=pl.BlockSpec((_BLK, 1), lambda i: (i, 0)),
    out_shape=jax.ShapeDtypeStruct((_BATCH, 1), jnp.float32),
)


def kernel(X, user_table, item_table, W1, b1, W2, b2):
    Xi = X.astype(jnp.int32)
    u_idx = Xi[:, 0].reshape(_BATCH // _CHUNK, _CHUNK)
    i_idx = Xi[:, 1].reshape(_BATCH // _CHUNK, _CHUNK)
    u_rows, i_rows = _sc_gather()(u_idx, i_idx, user_table, item_table)
    return _mlp(u_rows, i_rows, W1[:_ED], W1[_ED:],
                b1.reshape(1, _N_ACT), W2.reshape(1, _N_ACT),
                b2.reshape(1, 1))


# speedup vs baseline: 1.4991x; 1.4991x over previous
import functools

import jax
import jax.numpy as jnp
from jax import lax
from jax.experimental import pallas as pl
from jax.experimental.pallas import tpu as pltpu
from jax.experimental.pallas import tpu_sc as plsc

_BATCH = 16384
_ED = 64
_N_ACT = 128

_NC = 2
_NS = 16
_NW = _NC * _NS
_RPW = _BATCH // _NW
_CW = 128
_IDXR = _RPW // _CW
_GRP = 16


def _sc_gather_body(u_idx, i_idx, ut, it, u_out, i_out,
                    idx_v, stage, sem):
    wid = lax.axis_index("s") * _NC + lax.axis_index("c")
    crow = wid * _IDXR
    base = wid * _RPW

    def one_table(idx_hbm, tab, out_hbm):
        pltpu.sync_copy(idx_hbm.at[pl.ds(crow, _IDXR)], idx_v)

        def grp_body(g, _):
            vec = idx_v[g >> 3, pl.ds((g & 7) * _GRP, _GRP)]
            copies = [
                pltpu.async_copy(
                    tab.at[pl.ds(vec[k], 1)],
                    stage.at[pl.ds(g * _GRP + k, 1)],
                    sem,
                )
                for k in range(_GRP)
            ]
            for c in copies:
                c.wait()
            return 0

        lax.fori_loop(0, _RPW // _GRP, grp_body, 0)
        pltpu.sync_copy(stage, out_hbm.at[pl.ds(base, _RPW)])

    one_table(u_idx, ut, u_out)
    one_table(i_idx, it, i_out)


@functools.cache
def _sc_gather():
    return functools.partial(
        pl.kernel,
        out_type=[
            jax.ShapeDtypeStruct((_BATCH, _ED), jnp.float32),
            jax.ShapeDtypeStruct((_BATCH, _ED), jnp.float32),
        ],
        mesh=plsc.VectorSubcoreMesh(core_axis_name="c", subcore_axis_name="s"),
        scratch_types=[
            pltpu.VMEM((_IDXR, _CW), jnp.int32),
            pltpu.VMEM((_RPW, _ED), jnp.float32),
            pltpu.SemaphoreType.DMA,
        ],
    )(_sc_gather_body)


_BLK = 2048


def _mlp_body(u_ref, i_ref, w1u_ref, w1i_ref, b1_ref, w2_ref, b2_ref, o_ref):
    h = jnp.dot(u_ref[...], w1u_ref[...], preferred_element_type=jnp.float32)
    h += jnp.dot(i_ref[...], w1i_ref[...], preferred_element_type=jnp.float32)
    h = jnp.maximum(h + b1_ref[...], 0.0)
    p = jnp.sum(h * w2_ref[...], axis=1, keepdims=True) + b2_ref[...]
    o_ref[...] = 5.0 / (1.0 + jnp.exp(-p))


_mlp = pl.pallas_call(
    _mlp_body,
    grid=(_BATCH // _BLK,),
    in_specs=[
        pl.BlockSpec((_BLK, _ED), lambda i: (i, 0)),
        pl.BlockSpec((_BLK, _ED), lambda i: (i, 0)),
        pl.BlockSpec((_ED, _N_ACT), lambda i: (0, 0)),
        pl.BlockSpec((_ED, _N_ACT), lambda i: (0, 0)),
        pl.BlockSpec((1, _N_ACT), lambda i: (0, 0)),
        pl.BlockSpec((1, _N_ACT), lambda i: (0, 0)),
        pl.BlockSpec((1, 1), lambda i: (0, 0)),
    ],
    out_specs=pl.BlockSpec((_BLK, 1), lambda i: (i, 0)),
    out_shape=jax.ShapeDtypeStruct((_BATCH, 1), jnp.float32),
)


def kernel(X, user_table, item_table, W1, b1, W2, b2):
    Xi = X.astype(jnp.int32)
    u_idx = Xi[:, 0].reshape(_BATCH // _CW, _CW)
    i_idx = Xi[:, 1].reshape(_BATCH // _CW, _CW)
    u_rows, i_rows = _sc_gather()(u_idx, i_idx, user_table, item_table)
    return _mlp(u_rows, i_rows, W1[:_ED], W1[_ED:],
                b1.reshape(1, _N_ACT), W2.reshape(1, _N_ACT),
                b2.reshape(1, 1))
